# Initial kernel scaffold; baseline (speedup 1.0000x reference)
import jax
import jax.numpy as jnp
from jax.experimental import pallas as pl


def kernel(inputs, k, gate_w, gate_b, expert_w, expert_b):
    raise NotImplementedError("write your pallas kernel here")


# speedup vs baseline: 2.7528x; 2.7528x over previous
import functools

import jax
import jax.numpy as jnp
from jax.experimental import pallas as pl

_B, _C, _H, _W = 8, 96, 128, 128
_E = 8


def _moe_kernel(x_ref, k_ref, gw_ref, gb_ref, ew_ref, eb_ref, o_ref):
    b = pl.program_id(0)
    x = x_ref[...]
    pooled = jnp.mean(x, axis=(2, 3))
    logits = (
        jnp.dot(pooled, gw_ref[...], preferred_element_type=jnp.float32)
        + gb_ref[...]
    )
    w = jax.nn.softmax(logits, axis=-1)

    idx = jax.lax.broadcasted_iota(jnp.int32, (1, _E), 1)
    i1 = jnp.argmax(w, axis=1)[:, None]
    mask1 = idx == i1
    w_rest = jnp.where(mask1, -jnp.inf, w)
    i2 = jnp.argmax(w_rest, axis=1)[:, None]
    g = jnp.where(mask1 | (idx == i2), w, 0.0)

    scale = 1.0 + jnp.dot(g, ew_ref[...], preferred_element_type=jnp.float32)
    k_row = k_ref[pl.ds(b, 1), :]
    bias = (
        jnp.dot(g, eb_ref[...], preferred_element_type=jnp.float32)
        + jnp.sum(g) * k_row
    )
    o_ref[...] = x * scale[:, :, None, None] + bias[:, :, None, None]


@jax.jit
def kernel(inputs, k, gate_w, gate_b, expert_w, expert_b):
    k2 = k.reshape(_B, _C)
    gb2 = gate_b.reshape(1, _E)
    return pl.pallas_call(
        _moe_kernel,
        grid=(_B,),
        in_specs=[
            pl.BlockSpec((1, _C, _H, _W), lambda b: (b, 0, 0, 0)),
            pl.BlockSpec((_B, _C), lambda b: (0, 0)),
            pl.BlockSpec((_C, _E), lambda b: (0, 0)),
            pl.BlockSpec((1, _E), lambda b: (0, 0)),
            pl.BlockSpec((_E, _C), lambda b: (0, 0)),
            pl.BlockSpec((_E, _C), lambda b: (0, 0)),
        ],
        out_specs=pl.BlockSpec((1, _C, _H, _W), lambda b: (b, 0, 0, 0)),
        out_shape=jax.ShapeDtypeStruct((_B, _C, _H, _W), inputs.dtype),
    )(inputs, k2, gate_w, gb2, expert_w, expert_b)
